# baseline (speedup 1.0000x reference)
import jax
import jax.numpy as jnp
from jax.experimental import pallas as pl
from jax.experimental.pallas import tpu as pltpu

_N = 10000
_NPAD = 10240
_DT = 1024
_CH = 1024
_NT = _NPAD // _DT
_NCH = _NPAD // _CH
_THR = 0.3
_HID = 32


def _gnn_kernel(x_ref, w1_ref, b1_ref, w2_ref, b2_ref, out_ref,
                c_scr, ct_scr, ddr_scr, ddc_scr, v_scr, vc_scr,
                g_scr, gc_scr, deg_scr, pdeg_scr, pnum_scr):
    phase = pl.program_id(0)
    t = pl.program_id(1)
    d0 = t * _DT

    @pl.when((phase == 0) & (t == 0))
    def _init():
        flat = x_ref[...].reshape(-1, _NPAD)
        ctr = flat.T
        mu = jnp.mean(ctr, axis=1, keepdims=True)
        ct = ctr - mu
        ct_scr[...] = ct
        c_scr[...] = ct.T
        v = jnp.mean(x_ref[:, x_ref.shape[1] - 1, :], axis=0,
                     keepdims=True)
        v_scr[...] = v
        vc_scr[...] = jnp.broadcast_to(v, (8, _NPAD)).T[:, 0:1]

        def dchunk(i, _):
            s0 = i * _CH
            blk = jax.lax.dot_general(
                ct_scr[pl.ds(s0, _CH), :],
                c_scr[:, pl.ds(s0, _CH)],
                (((1,), (0,)), ((), ())),
                preferred_element_type=jnp.float32)
            eye = (jax.lax.broadcasted_iota(jnp.int32, (_CH, _CH), 0) ==
                   jax.lax.broadcasted_iota(jnp.int32, (_CH, _CH), 1))
            dz = jnp.where(eye, blk, 0.0)
            fact = jnp.float32(flat.shape[0] - 1)
            scale = jnp.sqrt(jnp.float32(_THR) * fact)
            ddr_scr[0:1, pl.ds(s0, _CH)] = scale * jnp.sqrt(
                jnp.sum(dz, axis=0, keepdims=True) / fact)
            ddc_scr[pl.ds(s0, _CH), 0:1] = scale * jnp.sqrt(
                jnp.sum(dz, axis=1, keepdims=True) / fact)
            return 0

        jax.lax.fori_loop(0, _NCH, dchunk, 0)

    @pl.when(t == 0)
    def _zero_acc():
        pdeg_scr[...] = jnp.zeros((1, _NPAD), jnp.float32)
        pnum_scr[...] = jnp.zeros((1, _NPAD), jnp.float32)

    cd = c_scr[:, pl.ds(d0, _DT)]
    ddd = ddr_scr[:, pl.ds(d0, _DT)]

    def block_cond(s0):
        cts = ct_scr[pl.ds(s0, _CH), :]
        s = jax.lax.dot_general(cts, cd, (((1,), (0,)), ((), ())),
                                preferred_element_type=jnp.float32)
        dds = ddc_scr[pl.ds(s0, _CH), :]
        return jnp.abs(s) > dds * ddd

    @pl.when(phase == 0)
    def _blocks0():
        def chunk(i, carry):
            @pl.when(i <= t)
            def _block():
                s0 = i * _CH
                mf = block_cond(s0).astype(jnp.float32)
                vsc = vc_scr[pl.ds(s0, _CH), :]
                pdeg_scr[0:1, pl.ds(d0, _DT)] += jnp.sum(
                    mf, axis=0, keepdims=True)
                pnum_scr[0:1, pl.ds(d0, _DT)] += jnp.sum(
                    mf * vsc, axis=0, keepdims=True)

                @pl.when(i < t)
                def _sym():
                    mft = mf.T
                    valc = vc_scr[pl.ds(d0, _DT), :]
                    pdeg_scr[0:1, pl.ds(s0, _CH)] += jnp.sum(
                        mft, axis=0, keepdims=True)
                    pnum_scr[0:1, pl.ds(s0, _CH)] += jnp.sum(
                        mft * valc, axis=0, keepdims=True)

            return carry

        jax.lax.fori_loop(0, _NCH, chunk, 0)

    @pl.when(phase == 1)
    def _blocks1():
        def chunk(i, carry):
            @pl.when(i <= t)
            def _block():
                s0 = i * _CH
                mf = block_cond(s0).astype(jnp.float32)
                gsc = gc_scr[pl.ds(s0, _CH), :]
                pnum_scr[0:1, pl.ds(d0, _DT)] += jnp.sum(
                    mf * gsc, axis=0, keepdims=True)

                @pl.when(i < t)
                def _sym():
                    mft = mf.T
                    gdc = gc_scr[pl.ds(d0, _DT), :]
                    pnum_scr[0:1, pl.ds(s0, _CH)] += jnp.sum(
                        mft * gdc, axis=0, keepdims=True)

            return carry

        jax.lax.fori_loop(0, _NCH, chunk, 0)

    @pl.when((phase == 0) & (t == _NT - 1))
    def _epilogue0():
        deg0 = pdeg_scr[...]
        num = pnum_scr[...]
        deg_scr[...] = deg0
        isolated = deg0 == 0.0
        a = jnp.where(isolated, v_scr[...],
                      num / jnp.where(isolated, 1.0, deg0))
        ab = jnp.broadcast_to(a, (_HID, _NPAD))
        h = jnp.maximum(ab * w1_ref[...] + b1_ref[...], 0.0)
        g = jnp.sum(h * w2_ref[...], axis=0, keepdims=True)
        g_scr[...] = g
        gc_scr[...] = jnp.broadcast_to(g, (8, _NPAD)).T[:, 0:1]

    @pl.when((phase == 1) & (t == _NT - 1))
    def _epilogue1():
        num = pnum_scr[...]
        deg0 = deg_scr[...]
        isolated = deg0 == 0.0
        out_ref[...] = jnp.where(
            isolated, g_scr[...],
            num / jnp.where(isolated, 1.0, deg0)) + b2_ref[...]


@jax.jit
def _run(xpad, w1, b1, w2, b2):
    return pl.pallas_call(
        _gnn_kernel,
        grid=(2, _NT),
        in_specs=[
            pl.BlockSpec(xpad.shape, lambda p, t: (0, 0, 0)),
            pl.BlockSpec((_HID, 1), lambda p, t: (0, 0)),
            pl.BlockSpec((_HID, 1), lambda p, t: (0, 0)),
            pl.BlockSpec((_HID, 1), lambda p, t: (0, 0)),
            pl.BlockSpec((1, 1), lambda p, t: (0, 0)),
        ],
        out_specs=pl.BlockSpec((1, _NPAD), lambda p, t: (0, 0)),
        out_shape=jax.ShapeDtypeStruct((1, _NPAD), jnp.float32),
        scratch_shapes=[
            pltpu.VMEM((64, _NPAD), jnp.float32),
            pltpu.VMEM((_NPAD, 64), jnp.float32),
            pltpu.VMEM((1, _NPAD), jnp.float32),
            pltpu.VMEM((_NPAD, 1), jnp.float32),
            pltpu.VMEM((1, _NPAD), jnp.float32),
            pltpu.VMEM((_NPAD, 1), jnp.float32),
            pltpu.VMEM((1, _NPAD), jnp.float32),
            pltpu.VMEM((_NPAD, 1), jnp.float32),
            pltpu.VMEM((1, _NPAD), jnp.float32),
            pltpu.VMEM((1, _NPAD), jnp.float32),
            pltpu.VMEM((1, _NPAD), jnp.float32),
        ],
        compiler_params=pltpu.CompilerParams(
            dimension_semantics=("arbitrary", "arbitrary"),
        ),
    )(xpad, w1, b1, w2, b2)


def kernel(x, W1, b1, W2, b2):
    B, S, N = x.shape
    xpad = jnp.pad(x, ((0, 0), (0, 0), (0, _NPAD - N)))
    row = _run(xpad,
               W1.reshape(1, _HID).T,
               b1.reshape(_HID, 1),
               W2.reshape(_HID, 1),
               b2.reshape(1, 1))
    gnn = row[0, :N]
    return jnp.broadcast_to(gnn[None, None, :], (B, S, N))


# speedup vs baseline: 1.2193x; 1.2193x over previous
import jax
import jax.numpy as jnp
from jax.experimental import pallas as pl
from jax.experimental.pallas import tpu as pltpu

_N = 10000
_NPAD = 10240
_DT = 1024
_CH = 1024
_NT = _NPAD // _DT
_NCH = _NPAD // _CH
_THR = 0.3
_HID = 32


def _gnn_kernel(x_ref, w1_ref, b1_ref, w2_ref, b2_ref, out_ref,
                c_scr, ct_scr, ddr_scr, ddc_scr, v_scr, vc_scr,
                g_scr, gc_scr, deg_scr, pdeg_scr, pnum_scr,
                dcol_scr, ncol_scr):
    phase = pl.program_id(0)
    t = pl.program_id(1)
    d0 = t * _DT

    @pl.when((phase == 0) & (t == 0))
    def _init():
        flat = x_ref[...].reshape(-1, _NPAD)
        ctr = flat.T
        mu = jnp.mean(ctr, axis=1, keepdims=True)
        ct = ctr - mu
        ct_scr[...] = ct
        c_scr[...] = ct.T
        v = jnp.mean(x_ref[:, x_ref.shape[1] - 1, :], axis=0,
                     keepdims=True)
        v_scr[...] = v
        vc_scr[...] = jnp.broadcast_to(v, (8, _NPAD)).T[:, 0:1]

        def dchunk(i, _):
            s0 = i * _CH
            blk = jax.lax.dot_general(
                ct_scr[pl.ds(s0, _CH), :],
                c_scr[:, pl.ds(s0, _CH)],
                (((1,), (0,)), ((), ())),
                preferred_element_type=jnp.float32)
            eye = (jax.lax.broadcasted_iota(jnp.int32, (_CH, _CH), 0) ==
                   jax.lax.broadcasted_iota(jnp.int32, (_CH, _CH), 1))
            dz = jnp.where(eye, blk, 0.0)
            fact = jnp.float32(flat.shape[0] - 1)
            scale = jnp.sqrt(jnp.float32(_THR) * fact)
            ddr_scr[0:1, pl.ds(s0, _CH)] = scale * jnp.sqrt(
                jnp.sum(dz, axis=0, keepdims=True) / fact)
            ddc_scr[pl.ds(s0, _CH), 0:1] = scale * jnp.sqrt(
                jnp.sum(dz, axis=1, keepdims=True) / fact)
            return 0

        jax.lax.fori_loop(0, _NCH, dchunk, 0)

    @pl.when(t == 0)
    def _zero_cols():
        dcol_scr[...] = jnp.zeros((_NPAD, 1), jnp.float32)
        ncol_scr[...] = jnp.zeros((_NPAD, 1), jnp.float32)

    cd = c_scr[:, pl.ds(d0, _DT)]
    ddd = ddr_scr[:, pl.ds(d0, _DT)]
    valr = jnp.where(phase == 0,
                     v_scr[0:1, pl.ds(d0, _DT)],
                     g_scr[0:1, pl.ds(d0, _DT)])

    pdeg_scr[0:1, pl.ds(d0, _DT)] = jnp.zeros((1, _DT), jnp.float32)
    pnum_scr[0:1, pl.ds(d0, _DT)] = jnp.zeros((1, _DT), jnp.float32)

    def block_cond(s0):
        cts = ct_scr[pl.ds(s0, _CH), :]
        s = jax.lax.dot_general(cts, cd, (((1,), (0,)), ((), ())),
                                preferred_element_type=jnp.float32)
        dds = ddc_scr[pl.ds(s0, _CH), :]
        return jnp.abs(s) > dds * ddd

    @pl.when(phase == 0)
    def _blocks0():
        def chunk(i, carry):
            @pl.when(i <= t)
            def _block():
                s0 = i * _CH
                mf = block_cond(s0).astype(jnp.float32)
                vsc = vc_scr[pl.ds(s0, _CH), :]
                pdeg_scr[0:1, pl.ds(d0, _DT)] += jnp.sum(
                    mf, axis=0, keepdims=True)
                pnum_scr[0:1, pl.ds(d0, _DT)] += jnp.sum(
                    mf * vsc, axis=0, keepdims=True)

                @pl.when(i < t)
                def _sym():
                    dcol_scr[pl.ds(s0, _CH), 0:1] += jnp.sum(
                        mf, axis=1, keepdims=True)
                    ncol_scr[pl.ds(s0, _CH), 0:1] += jnp.sum(
                        mf * valr, axis=1, keepdims=True)

            return carry

        jax.lax.fori_loop(0, _NCH, chunk, 0)

    @pl.when(phase == 1)
    def _blocks1():
        def chunk(i, carry):
            @pl.when(i <= t)
            def _block():
                s0 = i * _CH
                mf = block_cond(s0).astype(jnp.float32)
                gsc = gc_scr[pl.ds(s0, _CH), :]
                pnum_scr[0:1, pl.ds(d0, _DT)] += jnp.sum(
                    mf * gsc, axis=0, keepdims=True)

                @pl.when(i < t)
                def _sym():
                    ncol_scr[pl.ds(s0, _CH), 0:1] += jnp.sum(
                        mf * valr, axis=1, keepdims=True)

            return carry

        jax.lax.fori_loop(0, _NCH, chunk, 0)

    @pl.when((phase == 0) & (t == _NT - 1))
    def _epilogue0():
        dcol_row = jnp.broadcast_to(dcol_scr[...], (_NPAD, 8)).T[0:1, :]
        ncol_row = jnp.broadcast_to(ncol_scr[...], (_NPAD, 8)).T[0:1, :]
        deg0 = pdeg_scr[...] + dcol_row
        num = pnum_scr[...] + ncol_row
        deg_scr[...] = deg0
        isolated = deg0 == 0.0
        a = jnp.where(isolated, v_scr[...],
                      num / jnp.where(isolated, 1.0, deg0))
        ab = jnp.broadcast_to(a, (_HID, _NPAD))
        h = jnp.maximum(ab * w1_ref[...] + b1_ref[...], 0.0)
        g = jnp.sum(h * w2_ref[...], axis=0, keepdims=True)
        g_scr[...] = g
        gc_scr[...] = jnp.broadcast_to(g, (8, _NPAD)).T[:, 0:1]

    @pl.when((phase == 1) & (t == _NT - 1))
    def _epilogue1():
        ncol_row = jnp.broadcast_to(ncol_scr[...], (_NPAD, 8)).T[0:1, :]
        num = pnum_scr[...] + ncol_row
        deg0 = deg_scr[...]
        isolated = deg0 == 0.0
        out_ref[...] = jnp.where(
            isolated, g_scr[...],
            num / jnp.where(isolated, 1.0, deg0)) + b2_ref[...]


@jax.jit
def _run(xpad, w1, b1, w2, b2):
    return pl.pallas_call(
        _gnn_kernel,
        grid=(2, _NT),
        in_specs=[
            pl.BlockSpec(xpad.shape, lambda p, t: (0, 0, 0)),
            pl.BlockSpec((_HID, 1), lambda p, t: (0, 0)),
            pl.BlockSpec((_HID, 1), lambda p, t: (0, 0)),
            pl.BlockSpec((_HID, 1), lambda p, t: (0, 0)),
            pl.BlockSpec((1, 1), lambda p, t: (0, 0)),
        ],
        out_specs=pl.BlockSpec((1, _NPAD), lambda p, t: (0, 0)),
        out_shape=jax.ShapeDtypeStruct((1, _NPAD), jnp.float32),
        scratch_shapes=[
            pltpu.VMEM((64, _NPAD), jnp.float32),
            pltpu.VMEM((_NPAD, 64), jnp.float32),
            pltpu.VMEM((1, _NPAD), jnp.float32),
            pltpu.VMEM((_NPAD, 1), jnp.float32),
            pltpu.VMEM((1, _NPAD), jnp.float32),
            pltpu.VMEM((_NPAD, 1), jnp.float32),
            pltpu.VMEM((1, _NPAD), jnp.float32),
            pltpu.VMEM((_NPAD, 1), jnp.float32),
            pltpu.VMEM((1, _NPAD), jnp.float32),
            pltpu.VMEM((1, _NPAD), jnp.float32),
            pltpu.VMEM((1, _NPAD), jnp.float32),
            pltpu.VMEM((_NPAD, 1), jnp.float32),
            pltpu.VMEM((_NPAD, 1), jnp.float32),
        ],
        compiler_params=pltpu.CompilerParams(
            dimension_semantics=("arbitrary", "arbitrary"),
        ),
    )(xpad, w1, b1, w2, b2)


def kernel(x, W1, b1, W2, b2):
    B, S, N = x.shape
    xpad = jnp.pad(x, ((0, 0), (0, 0), (0, _NPAD - N)))
    row = _run(xpad,
               W1.reshape(1, _HID).T,
               b1.reshape(_HID, 1),
               W2.reshape(_HID, 1),
               b2.reshape(1, 1))
    gnn = row[0, :N]
    return jnp.broadcast_to(gnn[None, None, :], (B, S, N))
